# baseline (speedup 1.0000x reference)
import functools

import jax
import jax.numpy as jnp
from jax import lax
from jax.experimental import pallas as pl
from jax.experimental.pallas import tpu as pltpu
from jax.experimental.pallas import tpu_sc as plsc

B = 16
S = 16
R = B * S
V = 100000
P = 200
P_PAD = 224
PH = P_PAD // 2
VT = 51200
SV = V - VT
RPW = 8
NEG = -1e30


def _sc_partial(logits_hbm, seq_hbm, m2_hbm, d2_hbm, g_hbm,
                buf_a, buf_b, seq_row, idx_buf, g_buf, m_buf, d_buf,
                sem_a, sem_b, sem_g):
    cid = lax.axis_index("c")
    sid = lax.axis_index("s")
    w = sid * 2 + cid
    b = w // 2
    r0 = w * RPW

    pltpu.sync_copy(seq_hbm.at[pl.ds(b * P_PAD, P_PAD)], seq_row)

    for i in range(RPW):
        base = (r0 + i) * V

        def _mkidx(k, carry, i=i):
            idx_buf[pl.ds(i * P_PAD + k * 16, 16)] = (
                seq_row[pl.ds(k * 16, 16)] + base)
            return carry

        lax.fori_loop(0, P_PAD // 16, _mkidx, 0, unroll=True)

    gather_cps = []
    for i in range(RPW):
        for h in range(2):
            gather_cps.append(pltpu.async_copy(
                logits_hbm.at[idx_buf.at[pl.ds(i * P_PAD + h * PH, PH)]],
                g_buf.at[pl.ds(i * P_PAD + h * PH, PH)], sem_g))

    bufs = (buf_a, buf_b)
    sems = (sem_a, sem_b)

    def _fetch(i, buf, sem):
        return pltpu.async_copy(
            logits_hbm.at[pl.ds((r0 + i) * V + VT, SV)], buf, sem)

    cps = [None, None]
    cps[0] = _fetch(0, buf_a, sem_a)
    for i in range(RPW):
        cur = i % 2
        if i + 1 < RPW:
            cps[1 - cur] = _fetch(i + 1, bufs[1 - cur], sems[1 - cur])
        cps[cur].wait()
        buf = bufs[cur]

        def _maxbody(j, m):
            return jnp.maximum(m, buf[pl.ds(j * 16, 16)])

        m_vec = lax.fori_loop(0, SV // 16, _maxbody,
                              jnp.full((16,), NEG, jnp.float32), unroll=8)

        def _sumbody(j, acc):
            return acc + jnp.exp(buf[pl.ds(j * 16, 16)] - m_vec)

        d_vec = lax.fori_loop(0, SV // 16, _sumbody,
                              jnp.zeros((16,), jnp.float32), unroll=8)

        m_buf[pl.ds(i * 16, 16)] = m_vec
        d_buf[pl.ds(i * 16, 16)] = d_vec

    for cp in gather_cps:
        cp.wait()

    pltpu.sync_copy(m_buf, m2_hbm.at[pl.ds(w * RPW * 16, RPW * 16)])
    pltpu.sync_copy(d_buf, d2_hbm.at[pl.ds(w * RPW * 16, RPW * 16)])
    pltpu.sync_copy(g_buf, g_hbm.at[pl.ds(w * RPW * P_PAD, RPW * P_PAD)])


@functools.cache
def _sc_partial_call():
    mesh = plsc.VectorSubcoreMesh(core_axis_name="c", subcore_axis_name="s")
    return pl.kernel(
        _sc_partial,
        mesh=mesh,
        out_type=(
            jax.ShapeDtypeStruct((R * 16,), jnp.float32),
            jax.ShapeDtypeStruct((R * 16,), jnp.float32),
            jax.ShapeDtypeStruct((R * P_PAD,), jnp.float32),
        ),
        scratch_types=[
            pltpu.VMEM((SV,), jnp.float32),
            pltpu.VMEM((SV,), jnp.float32),
            pltpu.VMEM((P_PAD,), jnp.int32),
            pltpu.VMEM((RPW * P_PAD,), jnp.int32),
            pltpu.VMEM((RPW * P_PAD,), jnp.float32),
            pltpu.VMEM((RPW * 16,), jnp.float32),
            pltpu.VMEM((RPW * 16,), jnp.float32),
            pltpu.SemaphoreType.DMA,
            pltpu.SemaphoreType.DMA,
            pltpu.SemaphoreType.DMA,
        ],
        compiler_params=pltpu.CompilerParams(needs_layout_passes=False),
    )


def _tc_dense_body(logits_ref, m1_ref, d1_ref):
    x = logits_ref[0]
    m = jnp.max(x, axis=1, keepdims=True)
    d = jnp.sum(jnp.exp(x - m), axis=1, keepdims=True)
    m1_ref[0] = m
    d1_ref[0] = d


_tc_dense = pl.pallas_call(
    _tc_dense_body,
    grid=(B,),
    in_specs=[pl.BlockSpec((1, S, VT), lambda b: (b, 0, 0))],
    out_specs=[pl.BlockSpec((1, S, 1), lambda b: (b, 0, 0)),
               pl.BlockSpec((1, S, 1), lambda b: (b, 0, 0))],
    out_shape=[jax.ShapeDtypeStruct((B, S, 1), jnp.float32),
               jax.ShapeDtypeStruct((B, S, 1), jnp.float32)],
    compiler_params=pltpu.CompilerParams(
        dimension_semantics=("parallel",),
    ),
)


def _tc_combine_body(m1_ref, d1_ref, m2_ref, d2_ref, g_ref, seq_ref, out_ref):
    m1 = m1_ref[...]
    d1 = d1_ref[...]
    m2v = m2_ref[...]
    d2v = d2_ref[...]
    m2 = jnp.max(m2v, axis=2, keepdims=True)
    mm = jnp.maximum(m1, m2)
    d = (d1 * jnp.exp(m1 - mm)
         + jnp.sum(d2v * jnp.exp(m2v - mm), axis=2, keepdims=True))

    seq = seq_ref[...][:, 0, :]
    eq = seq[:, :, None] == seq[:, None, :]
    kk = lax.broadcasted_iota(jnp.int32, (1, P_PAD, P_PAD), 1)
    jj = lax.broadcasted_iota(jnp.int32, (1, P_PAD, P_PAD), 2)
    dup = jnp.any(eq & (jj < kk), axis=2)
    wts = ((seq != 0) & ~dup).astype(jnp.float32)

    n = jnp.sum(wts[:, None, :] * jnp.exp(g_ref[...] - mm),
                axis=2, keepdims=True)
    out_ref[...] = n / d


_tc_combine = pl.pallas_call(
    _tc_combine_body,
    grid=(1,),
    in_specs=[
        pl.BlockSpec((B, S, 1), lambda i: (0, 0, 0)),
        pl.BlockSpec((B, S, 1), lambda i: (0, 0, 0)),
        pl.BlockSpec((B, S, 16), lambda i: (0, 0, 0)),
        pl.BlockSpec((B, S, 16), lambda i: (0, 0, 0)),
        pl.BlockSpec((B, S, P_PAD), lambda i: (0, 0, 0)),
        pl.BlockSpec((B, 1, P_PAD), lambda i: (0, 0, 0)),
    ],
    out_specs=pl.BlockSpec((B, S, 1), lambda i: (0, 0, 0)),
    out_shape=jax.ShapeDtypeStruct((B, S, 1), jnp.float32),
)


def kernel(logits, penalty_sequence):
    seq = penalty_sequence.astype(jnp.int32)
    seq_p = jnp.pad(seq, ((0, 0), (0, P_PAD - P)))
    m2, d2, g = _sc_partial_call()(logits.reshape(-1), seq_p.reshape(-1))
    m1, d1 = _tc_dense(logits)
    out = _tc_combine(
        m1, d1,
        m2.reshape(B, S, 16), d2.reshape(B, S, 16),
        g.reshape(B, S, P_PAD), seq_p.reshape(B, 1, P_PAD))
    return out.reshape(B, S)


# speedup vs baseline: 1.5957x; 1.5957x over previous
import functools

import jax
import jax.numpy as jnp
from jax import lax
from jax.experimental import pallas as pl
from jax.experimental.pallas import tpu as pltpu
from jax.experimental.pallas import tpu_sc as plsc

B = 16
S = 16
R = B * S
V = 100000
P = 200
P_PAD = 224
NEG = -1e30

VT = 51200
VT2 = VT // 2
TAIL_BLK = 781
SC_LO = VT
SC_HI = 99968
SPAN = SC_HI - SC_LO
CH = 4096
NC = 12
RPW = 8

_MESH = dict(core_axis_name="c", subcore_axis_name="s")


def _worker_id():
    return lax.axis_index("s") * 2 + lax.axis_index("c")


def _sc_mask(seq_hbm, main_hbm, tail_hbm, buf, tbuf, seq_row):
    w = _worker_id()
    b = w // 2
    h = w % 2
    base_lo = h * VT2

    def _zero(i, c):
        buf[pl.ds(i * 16, 16)] = jnp.zeros((16,), jnp.float32)
        return c

    lax.fori_loop(0, VT2 // 16, _zero, 0, unroll=8)
    for i in range(128 // 16):
        tbuf[pl.ds(i * 16, 16)] = jnp.zeros((16,), jnp.float32)

    pltpu.sync_copy(seq_hbm.at[pl.ds(b * P_PAD, P_PAD)], seq_row)

    ones = jnp.ones((16,), jnp.float32)

    def _scatter(k, c):
        ids = seq_row[pl.ds(k * 16, 16)]
        vm = (ids != 0) & (ids >= base_lo) & (ids < base_lo + VT2)
        plsc.store_scatter(buf, [jnp.where(vm, ids - base_lo, 0)], ones,
                           mask=vm)
        vt = (ids >= SC_HI) & (ids < V)
        plsc.store_scatter(tbuf, [jnp.where(vt, ids - SC_HI, 0)], ones,
                           mask=vt)
        return c

    lax.fori_loop(0, P_PAD // 16, _scatter, 0, unroll=True)

    pltpu.sync_copy(buf, main_hbm.at[pl.ds(b * VT + base_lo, VT2)])
    pltpu.sync_copy(tbuf, tail_hbm.at[pl.ds(w * 128, 128)])


@functools.cache
def _sc_mask_call():
    return pl.kernel(
        _sc_mask,
        mesh=plsc.VectorSubcoreMesh(**_MESH),
        out_type=(
            jax.ShapeDtypeStruct((B * VT,), jnp.float32),
            jax.ShapeDtypeStruct((32 * 128,), jnp.float32),
        ),
        scratch_types=[
            pltpu.VMEM((VT2,), jnp.float32),
            pltpu.VMEM((128,), jnp.float32),
            pltpu.VMEM((P_PAD,), jnp.int32),
        ],
        compiler_params=pltpu.CompilerParams(needs_layout_passes=False),
    )


def _sc_stream(logits_hbm, seq_hbm, m2_hbm, d2_hbm, n2_hbm,
               buf_a, buf_b, mask_buf, seq_row, m_buf, d_buf, n_buf,
               sem_a, sem_b):
    w = _worker_id()
    b = w // 2
    s0 = (w % 2) * 8

    def _zero(i, c):
        mask_buf[pl.ds(i * 16, 16)] = jnp.zeros((16,), jnp.float32)
        return c

    lax.fori_loop(0, SPAN // 16, _zero, 0, unroll=8)
    pltpu.sync_copy(seq_hbm.at[pl.ds(b * P_PAD, P_PAD)], seq_row)
    ones = jnp.ones((16,), jnp.float32)

    def _scatter(k, c):
        ids = seq_row[pl.ds(k * 16, 16)]
        vm = (ids >= SC_LO) & (ids < SC_HI)
        plsc.store_scatter(mask_buf, [jnp.where(vm, ids - SC_LO, 0)], ones,
                           mask=vm)
        return c

    lax.fori_loop(0, P_PAD // 16, _scatter, 0, unroll=True)

    def _initacc(i, c):
        m_buf[pl.ds(i * 16, 16)] = jnp.full((16,), NEG, jnp.float32)
        d_buf[pl.ds(i * 16, 16)] = jnp.zeros((16,), jnp.float32)
        n_buf[pl.ds(i * 16, 16)] = jnp.zeros((16,), jnp.float32)
        return c

    lax.fori_loop(0, RPW, _initacc, 0)

    bufs = (buf_a, buf_b)
    sems = (sem_a, sem_b)
    LAST_FV0 = (SC_LO + (NC - 1) * CH - (SC_HI - CH)) // 16
    s0a = pl.multiple_of(s0, 8)

    def _col(cc):
        return pl.multiple_of(
            jnp.where(cc == NC - 1, SC_HI - CH, SC_LO + cc * CH), 128)

    pltpu.async_copy(
        logits_hbm.at[b, pl.ds(s0a, 8), pl.ds(SC_LO, CH)], buf_a, sem_a)
    pltpu.async_copy(
        logits_hbm.at[b, pl.ds(s0a, 8), pl.ds(SC_LO + CH, CH)], buf_b, sem_b)

    def _outer(j, carry):
        for k in range(2):
            cc = 2 * j + k
            buf, sem = bufs[k], sems[k]
            fv0 = jnp.where(cc == NC - 1, LAST_FV0, 0)
            mbase = _col(cc) - SC_LO
            pltpu.make_async_copy(
                logits_hbm.at[b, pl.ds(s0a, 8), pl.ds(SC_LO, CH)],
                buf, sem).wait()

            def _rows(i, c2):
                mv = m_buf[pl.ds(i * 16, 16)]

                def _p1(fv, cm):
                    v = jnp.where(fv >= fv0, buf[i, pl.ds(fv * 16, 16)], NEG)
                    return jnp.maximum(cm, v)

                cm = lax.fori_loop(0, CH // 16, _p1,
                                   jnp.full((16,), NEG, jnp.float32),
                                   unroll=4)
                m_new = jnp.maximum(mv, cm)
                sc = jnp.exp(mv - m_new)

                def _p2(fv, carry2):
                    dd, nn = carry2
                    e = jnp.where(
                        fv >= fv0,
                        jnp.exp(buf[i, pl.ds(fv * 16, 16)] - m_new), 0.0)
                    mk = mask_buf[pl.ds(mbase + fv * 16, 16)]
                    return (dd + e, nn + e * mk)

                dc, nc_ = lax.fori_loop(
                    0, CH // 16, _p2,
                    (jnp.zeros((16,), jnp.float32),
                     jnp.zeros((16,), jnp.float32)),
                    unroll=4)
                m_buf[pl.ds(i * 16, 16)] = m_new
                d_buf[pl.ds(i * 16, 16)] = d_buf[pl.ds(i * 16, 16)] * sc + dc
                n_buf[pl.ds(i * 16, 16)] = n_buf[pl.ds(i * 16, 16)] * sc + nc_
                return c2

            lax.fori_loop(0, RPW, _rows, 0)

            @pl.when(cc + 2 < NC)
            def _prefetch():
                pltpu.async_copy(
                    logits_hbm.at[b, pl.ds(s0a, 8), pl.ds(_col(cc + 2), CH)],
                    buf, sem)
        return carry

    lax.fori_loop(0, NC // 2, _outer, 0)

    pltpu.sync_copy(m_buf, m2_hbm.at[pl.ds(w * 128, 128)])
    pltpu.sync_copy(d_buf, d2_hbm.at[pl.ds(w * 128, 128)])
    pltpu.sync_copy(n_buf, n2_hbm.at[pl.ds(w * 128, 128)])


@functools.cache
def _sc_stream_call():
    return pl.kernel(
        _sc_stream,
        mesh=plsc.VectorSubcoreMesh(**_MESH),
        out_type=(
            jax.ShapeDtypeStruct((R * 16,), jnp.float32),
            jax.ShapeDtypeStruct((R * 16,), jnp.float32),
            jax.ShapeDtypeStruct((R * 16,), jnp.float32),
        ),
        scratch_types=[
            pltpu.VMEM((8, CH), jnp.float32),
            pltpu.VMEM((8, CH), jnp.float32),
            pltpu.VMEM((SPAN,), jnp.float32),
            pltpu.VMEM((P_PAD,), jnp.int32),
            pltpu.VMEM((RPW * 16,), jnp.float32),
            pltpu.VMEM((RPW * 16,), jnp.float32),
            pltpu.VMEM((RPW * 16,), jnp.float32),
            pltpu.SemaphoreType.DMA,
            pltpu.SemaphoreType.DMA,
        ],
        compiler_params=pltpu.CompilerParams(needs_layout_passes=False),
    )


def _tc_dense_body(logits_ref, ltail_ref, mmain_ref, mtail_ref,
                   m1_ref, d1_ref, n1_ref):
    x = logits_ref[0]
    lanes = lax.broadcasted_iota(jnp.int32, (1, 128), 1)
    valid = lanes < (V - SC_HI)
    xt = jnp.where(valid, ltail_ref[0], NEG)
    m = jnp.maximum(jnp.max(x, axis=1, keepdims=True),
                    jnp.max(xt, axis=1, keepdims=True))
    e = jnp.exp(x - m)
    et = jnp.exp(xt - m)
    mk = mmain_ref[...].reshape(1, VT)
    mkt = mtail_ref[...].reshape(1, 128)
    d1_ref[0] = (jnp.sum(e, axis=1, keepdims=True)
                 + jnp.sum(et, axis=1, keepdims=True))
    n1_ref[0] = (jnp.sum(e * mk, axis=1, keepdims=True)
                 + jnp.sum(et * mkt, axis=1, keepdims=True))
    m1_ref[0] = m


_tc_dense = pl.pallas_call(
    _tc_dense_body,
    grid=(B,),
    in_specs=[
        pl.BlockSpec((1, S, VT), lambda b: (b, 0, 0)),
        pl.BlockSpec((1, S, 128), lambda b: (b, 0, TAIL_BLK)),
        pl.BlockSpec((VT,), lambda b: (b,)),
        pl.BlockSpec((128,), lambda b: (2 * b,)),
    ],
    out_specs=[pl.BlockSpec((1, S, 1), lambda b: (b, 0, 0))] * 3,
    out_shape=[jax.ShapeDtypeStruct((B, S, 1), jnp.float32)] * 3,
    compiler_params=pltpu.CompilerParams(
        dimension_semantics=("parallel",),
    ),
)


def _tc_combine_body(m1_ref, d1_ref, n1_ref, m2_ref, d2_ref, n2_ref, out_ref):
    m1 = m1_ref[...]
    m2v = m2_ref[...]
    mm = jnp.maximum(m1, jnp.max(m2v, axis=2, keepdims=True))
    w1 = jnp.exp(m1 - mm)
    w2 = jnp.exp(m2v - mm)
    d = (d1_ref[...] * w1
         + jnp.sum(d2_ref[...] * w2, axis=2, keepdims=True))
    n = (n1_ref[...] * w1
         + jnp.sum(n2_ref[...] * w2, axis=2, keepdims=True))
    out_ref[...] = n / d


_tc_combine = pl.pallas_call(
    _tc_combine_body,
    grid=(1,),
    in_specs=[
        pl.BlockSpec((B, S, 1), lambda i: (0, 0, 0)),
        pl.BlockSpec((B, S, 1), lambda i: (0, 0, 0)),
        pl.BlockSpec((B, S, 1), lambda i: (0, 0, 0)),
        pl.BlockSpec((B, S, 16), lambda i: (0, 0, 0)),
        pl.BlockSpec((B, S, 16), lambda i: (0, 0, 0)),
        pl.BlockSpec((B, S, 16), lambda i: (0, 0, 0)),
    ],
    out_specs=pl.BlockSpec((B, S, 1), lambda i: (0, 0, 0)),
    out_shape=jax.ShapeDtypeStruct((B, S, 1), jnp.float32),
)


def kernel(logits, penalty_sequence):
    seq = penalty_sequence.astype(jnp.int32)
    seq_f = jnp.pad(seq, ((0, 0), (0, P_PAD - P))).reshape(-1)
    mask_main, mask_tail = _sc_mask_call()(seq_f)
    m2, d2, n2 = _sc_stream_call()(logits, seq_f)
    m1, d1, n1 = _tc_dense(logits, logits, mask_main, mask_tail)
    out = _tc_combine(m1, d1, n1,
                      m2.reshape(B, S, 16), d2.reshape(B, S, 16),
                      n2.reshape(B, S, 16))
    return out.reshape(B, S)


# speedup vs baseline: 2.6746x; 1.6761x over previous
import functools

import jax
import jax.numpy as jnp
from jax import lax
from jax.experimental import pallas as pl
from jax.experimental.pallas import tpu as pltpu
from jax.experimental.pallas import tpu_sc as plsc

B = 16
S = 16
V = 100000
P = 200
P_PAD = 208
VH = V // 2
NEG = -1e30

@functools.cache
def _sc_build_mask_call():
    mesh = plsc.VectorSubcoreMesh(core_axis_name="c", subcore_axis_name="s")
    return pl.kernel(
        _sc_build_mask,
        mesh=mesh,
        out_type=jax.ShapeDtypeStruct((B * V,), jnp.float32),
        scratch_types=[
            pltpu.VMEM((VH,), jnp.float32),
            pltpu.VMEM((P_PAD,), jnp.int32),
        ],
        compiler_params=pltpu.CompilerParams(needs_layout_passes=False),
    )


def _sc_build_mask(seq_hbm, mask_hbm, buf, seq_row):
    cid = lax.axis_index("c")
    sid = lax.axis_index("s")
    wid = sid * 2 + cid
    b = wid // 2
    half = wid % 2
    base_v = half * VH

    def _zero(i, carry):
        buf[pl.ds(i * 16, 16)] = jnp.zeros((16,), jnp.float32)
        return carry

    lax.fori_loop(0, VH // 16, _zero, 0, unroll=8)

    pltpu.sync_copy(seq_hbm.at[pl.ds(b * P_PAD, P_PAD)], seq_row)

    ones = jnp.ones((16,), jnp.float32)

    def _scatter(k, carry):
        ids = seq_row[pl.ds(k * 16, 16)]
        valid = (ids != 0) & (ids >= base_v) & (ids < base_v + VH)
        local = jnp.where(valid, ids - base_v, 0)
        plsc.store_scatter(buf, [local], ones, mask=valid)
        return carry

    lax.fori_loop(0, P_PAD // 16, _scatter, 0, unroll=True)

    pltpu.sync_copy(buf, mask_hbm.at[pl.ds(wid * VH, VH)])


VC = 100352


def _tc_body(logits_ref, mask_ref, out_ref):
    cols = lax.broadcasted_iota(jnp.int32, (1, VC), 1)
    valid = cols < V
    x = jnp.where(valid, logits_ref[0], NEG)
    m = jnp.max(x, axis=1, keepdims=True)
    e = jnp.exp(x - m)
    mk = jnp.where(valid, mask_ref[0], 0.0)
    d = jnp.sum(e, axis=1, keepdims=True)
    n = jnp.sum(e * mk, axis=1, keepdims=True)
    out_ref[0] = n / d


_tc_call = pl.pallas_call(
    _tc_body,
    grid=(B,),
    in_specs=[
        pl.BlockSpec((1, S, VC), lambda b: (b, 0, 0)),
        pl.BlockSpec((1, 1, VC), lambda b: (b, 0, 0)),
    ],
    out_specs=pl.BlockSpec((1, S, 1), lambda b: (b, 0, 0)),
    out_shape=jax.ShapeDtypeStruct((B, S, 1), jnp.float32),
    compiler_params=pltpu.CompilerParams(
        dimension_semantics=("parallel",),
    ),
)


def kernel(logits, penalty_sequence):
    seq = penalty_sequence.astype(jnp.int32)
    seq_p = jnp.pad(seq, ((0, 0), (0, P_PAD - P)))
    mask = _sc_build_mask_call()(seq_p.reshape(-1))
    mask3 = mask.reshape(B, 1, V)
    out = _tc_call(logits, mask3)
    return out.reshape(B, S)
